# Initial kernel scaffold; baseline (speedup 1.0000x reference)
import functools

import jax
import jax.numpy as jnp
from jax import lax
from jax.experimental import pallas as pl
from jax.experimental.pallas import tpu as pltpu
from jax.experimental.pallas import tpu_sc as plsc

VOCAB = 1000
D = 64
BATCH = 4096
SEQ = 200
N = BATCH * SEQ

NC = 2
NS = 16
NW = NC * NS
PER_W = N // NW

IVEC = 128
K = 8
CHUNK = K * IVEC
NCHUNK = PER_W // CHUNK

_mesh = plsc.VectorSubcoreMesh(core_axis_name="c", subcore_axis_name="s")


@functools.partial(
    pl.kernel,
    out_type=jax.ShapeDtypeStruct((N, D), jnp.float32),
    mesh=_mesh,
    scratch_types=[
        pltpu.VMEM((K, IVEC), jnp.int32),
        pltpu.VMEM((CHUNK, D), jnp.float32),
        pltpu.SemaphoreType.DMA,
    ],
)
def _embed_sc(table_hbm, idx_hbm, out_hbm, idx_v, rows_v, gsem):
    wid = lax.axis_index("s") * NC + lax.axis_index("c")
    row_base = wid * (PER_W // IVEC)
    base = wid * PER_W

    def body(i, _):
        pltpu.sync_copy(idx_hbm.at[pl.ds(row_base + i * K, K)], idx_v)
        copies = []
        for j in range(K):
            copies.append(
                pltpu.async_copy(
                    table_hbm.at[idx_v.at[j]],
                    rows_v.at[pl.ds(j * IVEC, IVEC)],
                    gsem,
                )
            )
        for c in copies:
            c.wait()
        pltpu.sync_copy(rows_v, out_hbm.at[pl.ds(base + i * CHUNK, CHUNK)])
        return ()

    lax.fori_loop(0, NCHUNK, body, ())


def kernel(table, inp):
    idx = inp.reshape(N // IVEC, IVEC)
    out = _embed_sc(table, idx)
    return out.reshape(BATCH, SEQ, D)


# speedup vs baseline: 3.5811x; 3.5811x over previous
import functools

import jax
import jax.numpy as jnp
from jax import lax
from jax.experimental import pallas as pl
from jax.experimental.pallas import tpu as pltpu
from jax.experimental.pallas import tpu_sc as plsc

VOCAB = 1000
D = 64
BATCH = 4096
SEQ = 200
N = BATCH * SEQ

NC = 2
NS = 16
NW = NC * NS
PER_W = N // NW

IVEC = 128
K = 8
CHUNK = K * IVEC
NCHUNK = PER_W // CHUNK

_mesh = plsc.VectorSubcoreMesh(core_axis_name="c", subcore_axis_name="s")


@functools.partial(
    pl.kernel,
    out_type=jax.ShapeDtypeStruct((N, D), jnp.float32),
    mesh=_mesh,
    scratch_types=[
        pltpu.VMEM((K, IVEC), jnp.int32),
        pltpu.VMEM((CHUNK, D), jnp.float32),
        pltpu.SemaphoreType.DMA,
    ],
    compiler_params=pltpu.CompilerParams(use_tc_tiling_on_sc=False),
)
def _embed_sc(table_hbm, idx_hbm, out_hbm, idx_v, rows_v, gsem):
    wid = lax.axis_index("s") * NC + lax.axis_index("c")
    row_base = wid * (PER_W // IVEC)
    base = wid * PER_W

    def body(i, _):
        pltpu.sync_copy(idx_hbm.at[pl.ds(row_base + i * K, K)], idx_v)
        copies = []
        for j in range(K):
            copies.append(
                pltpu.async_copy(
                    table_hbm.at[idx_v.at[j]],
                    rows_v.at[pl.ds(j * IVEC, IVEC)],
                    gsem,
                )
            )
        for c in copies:
            c.wait()
        pltpu.sync_copy(rows_v, out_hbm.at[pl.ds(base + i * CHUNK, CHUNK)])
        return ()

    lax.fori_loop(0, NCHUNK, body, ())


def kernel(table, inp):
    idx = inp.reshape(N // IVEC, IVEC)
    out = _embed_sc(table, idx)
    return out.reshape(BATCH, SEQ, D)
